# baseline (device time: 19000 ns/iter reference)
import jax
import jax.numpy as jnp
from jax import lax
from jax.experimental import pallas as pl
from jax.experimental.pallas import tpu as pltpu

NCHUNK = 1


def kernel(x):
    m, n = x.shape
    half = n // 2
    qm = m // 4
    rpc = qm // NCHUNK

    def body(x_ref, out_ref, stage, land, local_sem, stage_sem, land_sems,
             ysend, yrecv, xsend, xrecv, zsend, zrecv, dsend, drecv):
        my_x = lax.axis_index("x")
        my_y = lax.axis_index("y")
        my_z = lax.axis_index("z")
        other_y = 1 - my_y
        ypeer = (my_x, other_y, my_z)
        xpeer = (1 - my_x, my_y, my_z)
        zpeer = (my_x, my_y, 1 - my_z)
        dpeer = (1 - my_x, my_y, 1 - my_z)
        q = 2 * my_x + my_z

        stage_cp = pltpu.make_async_copy(
            x_ref.at[pl.ds(q * qm, qm), pl.ds(other_y * half, half)],
            stage,
            stage_sem,
        )
        stage_cp.start()
        local = pltpu.make_async_copy(
            x_ref.at[:, pl.ds(my_y * half, half)],
            out_ref.at[pl.ds(my_y * m, m)],
            local_sem,
        )
        local.start()

        barrier_sem = pltpu.get_barrier_semaphore()
        for nbr in (ypeer, xpeer, zpeer, dpeer):
            pl.semaphore_signal(
                barrier_sem, inc=1,
                device_id=nbr, device_id_type=pl.DeviceIdType.MESH,
            )
        pl.semaphore_wait(barrier_sem, 4)
        stage_cp.wait()

        y_rdmas = []
        for c in range(NCHUNK):
            rdma = pltpu.make_async_remote_copy(
                src_ref=stage.at[pl.ds(c * rpc, rpc)],
                dst_ref=land.at[pl.ds(c * rpc, rpc)],
                send_sem=ysend.at[c],
                recv_sem=yrecv.at[c],
                device_id=ypeer,
                device_id_type=pl.DeviceIdType.MESH,
            )
            rdma.start()
            y_rdmas.append(rdma)

        fwd_rdmas = []
        land_cps = []
        for c in range(NCHUNK):
            y_rdmas[c].wait_recv()
            row0 = other_y * m + q * qm + c * rpc
            for peer, ssem, rsem in (
                (xpeer, xsend, xrecv),
                (zpeer, zsend, zrecv),
                (dpeer, dsend, drecv),
            ):
                fwd = pltpu.make_async_remote_copy(
                    src_ref=land.at[pl.ds(c * rpc, rpc)],
                    dst_ref=out_ref.at[pl.ds(row0, rpc)],
                    send_sem=ssem.at[c],
                    recv_sem=rsem.at[c],
                    device_id=peer,
                    device_id_type=pl.DeviceIdType.MESH,
                )
                fwd.start()
                fwd_rdmas.append(fwd)
            cp = pltpu.make_async_copy(
                land.at[pl.ds(c * rpc, rpc)],
                out_ref.at[pl.ds(row0, rpc)],
                land_sems.at[c],
            )
            cp.start()
            land_cps.append(cp)

        for fwd in fwd_rdmas:
            fwd.wait_recv()
        for fwd in fwd_rdmas:
            fwd.wait_send()
        for c in range(NCHUNK):
            y_rdmas[c].wait_send()
            land_cps[c].wait()
        local.wait()

    return pl.pallas_call(
        body,
        out_shape=jax.ShapeDtypeStruct((2 * m, half), x.dtype),
        in_specs=[pl.BlockSpec(memory_space=pl.ANY)],
        out_specs=pl.BlockSpec(memory_space=pl.ANY),
        scratch_shapes=[
            pltpu.VMEM((qm, half), x.dtype),
            pltpu.VMEM((qm, half), x.dtype),
            pltpu.SemaphoreType.DMA,
            pltpu.SemaphoreType.DMA,
            pltpu.SemaphoreType.DMA((NCHUNK,)),
            pltpu.SemaphoreType.DMA((NCHUNK,)),
            pltpu.SemaphoreType.DMA((NCHUNK,)),
            pltpu.SemaphoreType.DMA((NCHUNK,)),
            pltpu.SemaphoreType.DMA((NCHUNK,)),
            pltpu.SemaphoreType.DMA((NCHUNK,)),
            pltpu.SemaphoreType.DMA((NCHUNK,)),
            pltpu.SemaphoreType.DMA((NCHUNK,)),
            pltpu.SemaphoreType.DMA((NCHUNK,)),
        ],
        compiler_params=pltpu.CompilerParams(collective_id=0),
    )(x)


# device time: 17176 ns/iter; 1.1062x vs baseline; 1.1062x over previous
import jax
import jax.numpy as jnp
from jax import lax
from jax.experimental import pallas as pl
from jax.experimental.pallas import tpu as pltpu

NCHUNK = 8


def kernel(x):
    m, n = x.shape
    half = n // 2
    qm = m // 4
    rpc = qm // NCHUNK

    def body(x_ref, out_ref, stage, land, local_sem, stage_sem, land_sems,
             ysend, yrecv, xsend, xrecv, zsend, zrecv, dsend, drecv):
        my_x = lax.axis_index("x")
        my_y = lax.axis_index("y")
        my_z = lax.axis_index("z")
        other_y = 1 - my_y
        ypeer = (my_x, other_y, my_z)
        xpeer = (1 - my_x, my_y, my_z)
        zpeer = (my_x, my_y, 1 - my_z)
        dpeer = (1 - my_x, my_y, 1 - my_z)
        q = 2 * my_x + my_z

        stage_cp = pltpu.make_async_copy(
            x_ref.at[pl.ds(q * qm, qm), pl.ds(other_y * half, half)],
            stage,
            stage_sem,
        )
        stage_cp.start()
        local = pltpu.make_async_copy(
            x_ref.at[:, pl.ds(my_y * half, half)],
            out_ref.at[pl.ds(my_y * m, m)],
            local_sem,
        )
        local.start()

        barrier_sem = pltpu.get_barrier_semaphore()
        for nbr in (ypeer, xpeer, zpeer, dpeer):
            pl.semaphore_signal(
                barrier_sem, inc=1,
                device_id=nbr, device_id_type=pl.DeviceIdType.MESH,
            )
        pl.semaphore_wait(barrier_sem, 4)
        stage_cp.wait()

        y_rdmas = []
        for c in range(NCHUNK):
            rdma = pltpu.make_async_remote_copy(
                src_ref=stage.at[pl.ds(c * rpc, rpc)],
                dst_ref=land.at[pl.ds(c * rpc, rpc)],
                send_sem=ysend.at[c],
                recv_sem=yrecv.at[c],
                device_id=ypeer,
                device_id_type=pl.DeviceIdType.MESH,
            )
            rdma.start()
            y_rdmas.append(rdma)

        fwd_rdmas = []
        land_cps = []
        for c in range(NCHUNK):
            y_rdmas[c].wait_recv()
            row0 = other_y * m + q * qm + c * rpc
            for peer, ssem, rsem in (
                (xpeer, xsend, xrecv),
                (zpeer, zsend, zrecv),
                (dpeer, dsend, drecv),
            ):
                fwd = pltpu.make_async_remote_copy(
                    src_ref=land.at[pl.ds(c * rpc, rpc)],
                    dst_ref=out_ref.at[pl.ds(row0, rpc)],
                    send_sem=ssem.at[c],
                    recv_sem=rsem.at[c],
                    device_id=peer,
                    device_id_type=pl.DeviceIdType.MESH,
                )
                fwd.start()
                fwd_rdmas.append(fwd)
            cp = pltpu.make_async_copy(
                land.at[pl.ds(c * rpc, rpc)],
                out_ref.at[pl.ds(row0, rpc)],
                land_sems.at[c],
            )
            cp.start()
            land_cps.append(cp)

        for fwd in fwd_rdmas:
            fwd.wait_recv()
        for fwd in fwd_rdmas:
            fwd.wait_send()
        for c in range(NCHUNK):
            y_rdmas[c].wait_send()
            land_cps[c].wait()
        local.wait()

    return pl.pallas_call(
        body,
        out_shape=jax.ShapeDtypeStruct((2 * m, half), x.dtype),
        in_specs=[pl.BlockSpec(memory_space=pl.ANY)],
        out_specs=pl.BlockSpec(memory_space=pl.ANY),
        scratch_shapes=[
            pltpu.VMEM((qm, half), x.dtype),
            pltpu.VMEM((qm, half), x.dtype),
            pltpu.SemaphoreType.DMA,
            pltpu.SemaphoreType.DMA,
            pltpu.SemaphoreType.DMA((NCHUNK,)),
            pltpu.SemaphoreType.DMA((NCHUNK,)),
            pltpu.SemaphoreType.DMA((NCHUNK,)),
            pltpu.SemaphoreType.DMA((NCHUNK,)),
            pltpu.SemaphoreType.DMA((NCHUNK,)),
            pltpu.SemaphoreType.DMA((NCHUNK,)),
            pltpu.SemaphoreType.DMA((NCHUNK,)),
            pltpu.SemaphoreType.DMA((NCHUNK,)),
            pltpu.SemaphoreType.DMA((NCHUNK,)),
        ],
        compiler_params=pltpu.CompilerParams(collective_id=0),
    )(x)


# device time: 14319 ns/iter; 1.3269x vs baseline; 1.1995x over previous
import jax
import jax.numpy as jnp
from jax import lax
from jax.experimental import pallas as pl
from jax.experimental.pallas import tpu as pltpu

NCHUNK = 4


def kernel(x):
    m, n = x.shape
    half = n // 2
    qm = m // 4
    rpc = qm // NCHUNK

    def body(x_ref, out_ref, stage, land, local_sem, stage_sem, land_sems,
             ysend, yrecv, dsend, drecv, xsend, xrecv, zsend, zrecv):
        my_x = lax.axis_index("x")
        my_y = lax.axis_index("y")
        my_z = lax.axis_index("z")
        other_y = 1 - my_y
        ypeer = (my_x, other_y, my_z)
        xpeer = (1 - my_x, my_y, my_z)
        zpeer = (my_x, my_y, 1 - my_z)
        q = 2 * my_x + my_z
        d = 2 * (1 - my_x) + (1 - my_z)

        stage_cp = pltpu.make_async_copy(
            x_ref.at[pl.ds(q * qm, qm), pl.ds(other_y * half, half)],
            stage,
            stage_sem,
        )
        stage_cp.start()
        local = pltpu.make_async_copy(
            x_ref.at[:, pl.ds(my_y * half, half)],
            out_ref.at[pl.ds(my_y * m, m)],
            local_sem,
        )
        local.start()

        barrier_sem = pltpu.get_barrier_semaphore()
        for nbr in (ypeer, xpeer, zpeer):
            pl.semaphore_signal(
                barrier_sem, inc=1,
                device_id=nbr, device_id_type=pl.DeviceIdType.MESH,
            )
        pl.semaphore_wait(barrier_sem, 3)
        stage_cp.wait()

        y_rdmas = []
        for c in range(NCHUNK):
            rdma = pltpu.make_async_remote_copy(
                src_ref=stage.at[pl.ds(c * rpc, rpc)],
                dst_ref=land.at[pl.ds(c * rpc, rpc)],
                send_sem=ysend.at[c],
                recv_sem=yrecv.at[c],
                device_id=ypeer,
                device_id_type=pl.DeviceIdType.MESH,
            )
            rdma.start()
            y_rdmas.append(rdma)
        d_rdmas = []
        for c in range(NCHUNK):
            rdma = pltpu.make_async_remote_copy(
                src_ref=x_ref.at[pl.ds(d * qm + c * rpc, rpc),
                                 pl.ds(other_y * half, half)],
                dst_ref=out_ref.at[pl.ds(my_y * m + d * qm + c * rpc, rpc)],
                send_sem=dsend.at[c],
                recv_sem=drecv.at[c],
                device_id=ypeer,
                device_id_type=pl.DeviceIdType.MESH,
            )
            rdma.start()
            d_rdmas.append(rdma)

        fwd_rdmas = []
        land_cps = []
        for c in range(NCHUNK):
            y_rdmas[c].wait_recv()
            row0 = other_y * m + q * qm + c * rpc
            for peer, ssem, rsem in (
                (xpeer, xsend, xrecv),
                (zpeer, zsend, zrecv),
            ):
                fwd = pltpu.make_async_remote_copy(
                    src_ref=land.at[pl.ds(c * rpc, rpc)],
                    dst_ref=out_ref.at[pl.ds(row0, rpc)],
                    send_sem=ssem.at[c],
                    recv_sem=rsem.at[c],
                    device_id=peer,
                    device_id_type=pl.DeviceIdType.MESH,
                )
                fwd.start()
                fwd_rdmas.append(fwd)
            cp = pltpu.make_async_copy(
                land.at[pl.ds(c * rpc, rpc)],
                out_ref.at[pl.ds(row0, rpc)],
                land_sems.at[c],
            )
            cp.start()
            land_cps.append(cp)

        for fwd in fwd_rdmas:
            fwd.wait_recv()
        for c in range(NCHUNK):
            d_rdmas[c].wait_recv()
        for fwd in fwd_rdmas:
            fwd.wait_send()
        for c in range(NCHUNK):
            y_rdmas[c].wait_send()
            d_rdmas[c].wait_send()
            land_cps[c].wait()
        local.wait()

    return pl.pallas_call(
        body,
        out_shape=jax.ShapeDtypeStruct((2 * m, half), x.dtype),
        in_specs=[pl.BlockSpec(memory_space=pl.ANY)],
        out_specs=pl.BlockSpec(memory_space=pl.ANY),
        scratch_shapes=[
            pltpu.VMEM((qm, half), x.dtype),
            pltpu.VMEM((qm, half), x.dtype),
            pltpu.SemaphoreType.DMA,
            pltpu.SemaphoreType.DMA,
            pltpu.SemaphoreType.DMA((NCHUNK,)),
            pltpu.SemaphoreType.DMA((NCHUNK,)),
            pltpu.SemaphoreType.DMA((NCHUNK,)),
            pltpu.SemaphoreType.DMA((NCHUNK,)),
            pltpu.SemaphoreType.DMA((NCHUNK,)),
            pltpu.SemaphoreType.DMA((NCHUNK,)),
            pltpu.SemaphoreType.DMA((NCHUNK,)),
            pltpu.SemaphoreType.DMA((NCHUNK,)),
            pltpu.SemaphoreType.DMA((NCHUNK,)),
        ],
        compiler_params=pltpu.CompilerParams(collective_id=0),
    )(x)
